# baseline (speedup 1.0000x reference)
import functools

import numpy as np
import jax
import jax.numpy as jnp
from jax import lax
from jax.experimental import pallas as pl
from jax.experimental.pallas import tpu as pltpu
from jax.experimental.pallas import tpu_sc as plsc

N_BINS = 64
INPUT_DIM = 64
R = 1.2
BOUND = 50.0

NC = 2
NS = 16
NW = NC * NS
CR = 512

_LOG2_POLY = (
    -0.02512326007229515, 0.2700380630042624, -1.2479651030208252,
    3.2494723906595535, -5.301716265875989, 6.089900348117342,
    -3.0346040497219526,
)
_LN2 = float(np.log(2.0))


def _log2_estrin(mant, e_f, scale, bias):
    c6, c5, c4, c3, c2, c1, c0 = [c * scale for c in _LOG2_POLY]
    c0 = c0 + bias * scale
    m2 = mant * mant
    t1 = c1 * mant + c0
    t2 = c3 * mant + c2
    t3 = c5 * mant + c4
    t4 = c6 * m2 + t3
    return (t4 * m2 + t2) * m2 + t1 + e_f * scale


def _static_mesh():
    m = N_BINS / 2
    x1L_raw = BOUND * (R - 1.0) / (np.power(R, m) - 1.0)
    index = np.arange(0, N_BINS + 1, dtype=np.float64).reshape(-1, 1) - m
    xr = np.where(index >= 0, (1.0 - np.power(R, index)) / (1.0 - R),
                  (1.0 - np.power(R, np.abs(index))) / (1.0 - R))
    xr = np.where(index >= 0, x1L_raw * xr, -x1L_raw * xr)
    xr = (xr + BOUND) / 2.0 / BOUND
    mesh = np.concatenate([np.array([[0.0]]), xr[1:-1, :1],
                           np.array([[1.0]])], 0).astype(np.float32)
    elmt = (mesh[1:] - mesh[:-1]).astype(np.float32)
    return mesh, elmt, x1L_raw


_MESH_NP, _ELMT_NP, _X1L_RAW = _static_mesh()
_T_SCALE = np.float32((R - 1.0) / _X1L_RAW)
_INV_L2R = np.float32(1.0 / np.log2(R))

_LUT_NP = ((np.log2(1.0 + (np.arange(1024, dtype=np.float64) + 0.5) / 1024.0)
            - 127.0) / np.log2(R)).astype(np.float32)


def _coeff_tables(p):
    elmt = jnp.asarray(_ELMT_NP)
    p0 = jnp.ones((1, INPUT_DIM), dtype=jnp.float32)
    px = jnp.exp(p) * (elmt[:-1] + elmt[1:]) / 2.0
    px = (1.0 - elmt[0]) / jnp.sum(px, 0, keepdims=True)
    px = px * jnp.exp(p)
    pdf = jnp.concatenate([p0, px, p0], 0)
    cell = (pdf[:-1, :] + pdf[1:, :]) / 2.0 * elmt
    F_ref = jnp.concatenate([jnp.zeros((1, INPUT_DIM), jnp.float32),
                             jnp.cumsum(cell, axis=0)[:N_BINS - 1]], 0)

    v1 = pdf[:N_BINS]
    v2 = pdf[1:N_BINS + 1]
    h = jnp.asarray(_ELMT_NP[:, 0])
    m0x = jnp.asarray(_MESH_NP[:N_BINS, 0] * (2.0 * BOUND) - BOUND)
    c = (v2 - v1) / ((2.0 * 2.0 * BOUND) * h)[:, None]
    b = v1 - (2.0 * m0x)[:, None] * c
    a = ((2.0 * BOUND) * F_ref - BOUND
         + (m0x * m0x)[:, None] * c - m0x[:, None] * v1)
    zrow = jnp.zeros((1, INPUT_DIM), jnp.float32)
    orow = jnp.ones((1, INPUT_DIM), jnp.float32)
    a = jnp.concatenate([zrow, a, zrow], 0)
    b = jnp.concatenate([orow, b, orow], 0)
    c = jnp.concatenate([zrow, c, zrow], 0)
    return a, b, c


def _split_float(a):
    bi = plsc.bitcast(a, jnp.int32)
    e_f = (bi >> 23).astype(jnp.float32)
    mant = plsc.bitcast((bi & 0x007FFFFF) | 0x3F800000, jnp.float32)
    return e_f, mant


def _lnf(a):
    e_f, mant = _split_float(a)
    return _log2_estrin(mant, e_f, _LN2, -127.0)


def _sc_body(batch, x_hbm, a_hbm, b_hbm, c_hbm, l_hbm, y_hbm, ld_hbm,
             xb, yb, ldb, ta, tb, tc, tl):
    rows_w = batch // NW
    nch = rows_w // CR
    wid = lax.axis_index("s") * NC + lax.axis_index("c")
    pltpu.sync_copy(a_hbm, ta)
    pltpu.sync_copy(b_hbm, tb)
    pltpu.sync_copy(c_hbm, tc)
    pltpu.sync_copy(l_hbm, tl)
    base = wid * rows_w
    lanes = lax.iota(jnp.int32, 16)
    ones = jnp.ones((16,), dtype=jnp.float32)

    def elem(xv, d):
        t = jnp.abs(xv) * _T_SCALE + 1.0
        bi = plsc.bitcast(t, jnp.int32)
        li = (bi >> 13) & 0x3FF
        e_f = (bi >> 23).astype(jnp.float32)
        mf = e_f * _INV_L2R + plsc.load_gather(tl, [li])
        mi = mf.astype(jnp.int32)
        mc = jnp.minimum(jnp.maximum(mi, 0), N_BINS // 2)
        neg = plsc.bitcast(xv, jnp.int32) >> 31
        k = (mc ^ neg) + (N_BINS // 2 + 1)
        tix = k * INPUT_DIM + d
        av = plsc.load_gather(ta, [tix])
        bv = plsc.load_gather(tb, [tix])
        cv = plsc.load_gather(tc, [tix])
        cx = cv * xv
        inner = bv + cx
        y = av + xv * inner
        g = inner + cx
        return y, g

    def chunk_body(ci, carry):
        r0 = base + ci * CR
        h0 = pl.multiple_of(r0 // 2, 8)
        pltpu.sync_copy(x_hbm.at[pl.ds(h0, CR // 2)], xb)

        @plsc.parallel_loop(0, CR // 16)
        def rg_body(rg):
            fbase = (rg * 16 + lanes) * INPUT_DIM

            @plsc.parallel_loop(0, INPUT_DIM // 4, unroll=2,
                                carry=(ones, ones, ones, ones))
            def d_body(i, accs):
                a0, a1, a2, a3 = accs
                out = []
                for j, acc in zip(range(4), (a0, a1, a2, a3)):
                    dv = (i * 4 + j + lanes) & (INPUT_DIM - 1)
                    fidx = fbase + dv
                    fr, fc = fidx >> 7, fidx & 127
                    xv = plsc.load_gather(xb, [fr, fc])
                    y, g = elem(xv, dv)
                    plsc.store_scatter(yb, [fr, fc], y)
                    out.append(acc * g)
                return tuple(out)

            a0, a1, a2, a3 = d_body
            ld = _lnf(a0) + _lnf(a1) + _lnf(a2) + _lnf(a3)
            ldb[pl.ds(rg * 16, 16)] = ld

        pltpu.sync_copy(yb, y_hbm.at[pl.ds(h0, CR // 2)])
        pltpu.sync_copy(ldb, ld_hbm.at[pl.ds(r0, CR)])
        return carry

    lax.fori_loop(0, nch, chunk_body, 0)


def kernel(x, p):
    batch = x.shape[0]
    a, b, c = _coeff_tables(p)
    mesh = plsc.VectorSubcoreMesh(core_axis_name="c", subcore_axis_name="s")
    run = pl.kernel(
        functools.partial(_sc_body, batch),
        out_type=(jax.ShapeDtypeStruct((batch * INPUT_DIM // 128, 128),
                                       jnp.float32),
                  jax.ShapeDtypeStruct((batch,), jnp.float32)),
        mesh=mesh,
        compiler_params=pltpu.CompilerParams(needs_layout_passes=False),
        scratch_types=[
            pltpu.VMEM((CR * INPUT_DIM // 128, 128), jnp.float32),
            pltpu.VMEM((CR * INPUT_DIM // 128, 128), jnp.float32),
            pltpu.VMEM((CR,), jnp.float32),
            pltpu.VMEM(((N_BINS + 2) * INPUT_DIM,), jnp.float32),
            pltpu.VMEM(((N_BINS + 2) * INPUT_DIM,), jnp.float32),
            pltpu.VMEM(((N_BINS + 2) * INPUT_DIM,), jnp.float32),
            pltpu.VMEM((1024,), jnp.float32),
        ],
    )
    yf, ld = run(x.reshape(batch * INPUT_DIM // 128, 128),
                 a.reshape(-1), b.reshape(-1), c.reshape(-1),
                 jnp.asarray(_LUT_NP))
    return (yf.reshape(batch, INPUT_DIM), ld)


# speedup vs baseline: 1.1189x; 1.1189x over previous
import functools

import numpy as np
import jax
import jax.numpy as jnp
from jax import lax
from jax.experimental import pallas as pl
from jax.experimental.pallas import tpu as pltpu
from jax.experimental.pallas import tpu_sc as plsc

N_BINS = 64
INPUT_DIM = 64
R = 1.2
BOUND = 50.0

NC = 2
NS = 16
NW = NC * NS
CR = 256

_LOG2_POLY = (
    -0.02512326007229515, 0.2700380630042624, -1.2479651030208252,
    3.2494723906595535, -5.301716265875989, 6.089900348117342,
    -3.0346040497219526,
)
_LN2 = float(np.log(2.0))


def _log2_estrin(mant, e_f, scale, bias):
    c6, c5, c4, c3, c2, c1, c0 = [c * scale for c in _LOG2_POLY]
    c0 = c0 + bias * scale
    m2 = mant * mant
    t1 = c1 * mant + c0
    t2 = c3 * mant + c2
    t3 = c5 * mant + c4
    t4 = c6 * m2 + t3
    return (t4 * m2 + t2) * m2 + t1 + e_f * scale


def _static_mesh():
    m = N_BINS / 2
    x1L_raw = BOUND * (R - 1.0) / (np.power(R, m) - 1.0)
    index = np.arange(0, N_BINS + 1, dtype=np.float64).reshape(-1, 1) - m
    xr = np.where(index >= 0, (1.0 - np.power(R, index)) / (1.0 - R),
                  (1.0 - np.power(R, np.abs(index))) / (1.0 - R))
    xr = np.where(index >= 0, x1L_raw * xr, -x1L_raw * xr)
    xr = (xr + BOUND) / 2.0 / BOUND
    mesh = np.concatenate([np.array([[0.0]]), xr[1:-1, :1],
                           np.array([[1.0]])], 0).astype(np.float32)
    elmt = (mesh[1:] - mesh[:-1]).astype(np.float32)
    return mesh, elmt, x1L_raw


_MESH_NP, _ELMT_NP, _X1L_RAW = _static_mesh()
_T_SCALE = np.float32((R - 1.0) / _X1L_RAW)
_INV_L2R = np.float32(1.0 / np.log2(R))

_LUT_NP = ((np.log2(1.0 + (np.arange(1024, dtype=np.float64) + 0.5) / 1024.0)
            - 127.0) / np.log2(R)).astype(np.float32)


def _coeff_tables(p):
    elmt = jnp.asarray(_ELMT_NP)
    p0 = jnp.ones((1, INPUT_DIM), dtype=jnp.float32)
    px = jnp.exp(p) * (elmt[:-1] + elmt[1:]) / 2.0
    px = (1.0 - elmt[0]) / jnp.sum(px, 0, keepdims=True)
    px = px * jnp.exp(p)
    pdf = jnp.concatenate([p0, px, p0], 0)
    cell = (pdf[:-1, :] + pdf[1:, :]) / 2.0 * elmt
    F_ref = jnp.concatenate([jnp.zeros((1, INPUT_DIM), jnp.float32),
                             jnp.cumsum(cell, axis=0)[:N_BINS - 1]], 0)

    v1 = pdf[:N_BINS]
    v2 = pdf[1:N_BINS + 1]
    h = jnp.asarray(_ELMT_NP[:, 0])
    m0x = jnp.asarray(_MESH_NP[:N_BINS, 0] * (2.0 * BOUND) - BOUND)
    c = (v2 - v1) / ((2.0 * 2.0 * BOUND) * h)[:, None]
    b = v1 - (2.0 * m0x)[:, None] * c
    a = ((2.0 * BOUND) * F_ref - BOUND
         + (m0x * m0x)[:, None] * c - m0x[:, None] * v1)
    zrow = jnp.zeros((1, INPUT_DIM), jnp.float32)
    orow = jnp.ones((1, INPUT_DIM), jnp.float32)
    a = jnp.concatenate([zrow, a, zrow], 0)
    b = jnp.concatenate([orow, b, orow], 0)
    c = jnp.concatenate([zrow, c, zrow], 0)
    return a, b, c


def _split_float(a):
    bi = plsc.bitcast(a, jnp.int32)
    e_f = (bi >> 23).astype(jnp.float32)
    mant = plsc.bitcast((bi & 0x007FFFFF) | 0x3F800000, jnp.float32)
    return e_f, mant


def _lnf(a):
    e_f, mant = _split_float(a)
    return _log2_estrin(mant, e_f, _LN2, -127.0)


def _sc_body(batch, x_hbm, a_hbm, b_hbm, c_hbm, l_hbm, y_hbm, ld_hbm,
             xb, yb, ldb, ta, tb, tc, tl):
    rows_w = batch // NW
    nch = rows_w // CR
    wid = lax.axis_index("s") * NC + lax.axis_index("c")
    pltpu.sync_copy(a_hbm, ta)
    pltpu.sync_copy(b_hbm, tb)
    pltpu.sync_copy(c_hbm, tc)
    pltpu.sync_copy(l_hbm, tl)
    base = wid * rows_w
    lanes = lax.iota(jnp.int32, 16)
    ones = jnp.ones((16,), dtype=jnp.float32)

    def elem(xv, d):
        t = jnp.abs(xv) * _T_SCALE + 1.0
        bi = plsc.bitcast(t, jnp.int32)
        li = (bi >> 13) & 0x3FF
        e_f = (bi >> 23).astype(jnp.float32)
        mf = e_f * _INV_L2R + plsc.load_gather(tl, [li])
        mi = mf.astype(jnp.int32)
        mc = jnp.minimum(jnp.maximum(mi, 0), N_BINS // 2)
        neg = plsc.bitcast(xv, jnp.int32) >> 31
        k = (mc ^ neg) + (N_BINS // 2 + 1)
        tix = k * INPUT_DIM + d
        av = plsc.load_gather(ta, [tix])
        bv = plsc.load_gather(tb, [tix])
        cv = plsc.load_gather(tc, [tix])
        cx = cv * xv
        inner = bv + cx
        y = av + xv * inner
        g = inner + cx
        return y, g

    def chunk_body(ci, carry):
        r0 = pl.multiple_of(base + ci * CR, 8)
        pltpu.sync_copy(x_hbm.at[pl.ds(r0, CR)], xb)

        @plsc.parallel_loop(0, CR // 16)
        def rg_body(rg):
            rows = rg * 16 + lanes

            @plsc.parallel_loop(0, INPUT_DIM // 4, unroll=2,
                                carry=(ones, ones, ones, ones))
            def d_body(i, accs):
                a0, a1, a2, a3 = accs
                out = []
                for j, acc in zip(range(4), (a0, a1, a2, a3)):
                    dv = (i * 4 + j + lanes) & (INPUT_DIM - 1)
                    xv = plsc.load_gather(xb, [rows, dv])
                    y, g = elem(xv, dv)
                    plsc.store_scatter(yb, [rows, dv], y)
                    out.append(acc * g)
                return tuple(out)

            a0, a1, a2, a3 = d_body
            ld = _lnf(a0) + _lnf(a1) + _lnf(a2) + _lnf(a3)
            ldb[pl.ds(rg * 16, 16)] = ld

        pltpu.sync_copy(yb, y_hbm.at[pl.ds(r0, CR)])
        pltpu.sync_copy(ldb, ld_hbm.at[pl.ds(r0, CR)])
        return carry

    lax.fori_loop(0, nch, chunk_body, 0)


def kernel(x, p):
    batch = x.shape[0]
    a, b, c = _coeff_tables(p)
    mesh = plsc.VectorSubcoreMesh(core_axis_name="c", subcore_axis_name="s")
    run = pl.kernel(
        functools.partial(_sc_body, batch),
        out_type=(jax.ShapeDtypeStruct((batch, INPUT_DIM), jnp.float32),
                  jax.ShapeDtypeStruct((batch,), jnp.float32)),
        mesh=mesh,
        compiler_params=pltpu.CompilerParams(needs_layout_passes=False),
        scratch_types=[
            pltpu.VMEM((CR, INPUT_DIM), jnp.float32),
            pltpu.VMEM((CR, INPUT_DIM), jnp.float32),
            pltpu.VMEM((CR,), jnp.float32),
            pltpu.VMEM(((N_BINS + 2) * INPUT_DIM,), jnp.float32),
            pltpu.VMEM(((N_BINS + 2) * INPUT_DIM,), jnp.float32),
            pltpu.VMEM(((N_BINS + 2) * INPUT_DIM,), jnp.float32),
            pltpu.VMEM((1024,), jnp.float32),
        ],
    )
    y, ld = run(x, a.reshape(-1), b.reshape(-1), c.reshape(-1),
                jnp.asarray(_LUT_NP))
    return (y, ld)


# speedup vs baseline: 1.3783x; 1.2319x over previous
import functools

import numpy as np
import jax
import jax.numpy as jnp
from jax import lax
from jax.experimental import pallas as pl
from jax.experimental.pallas import tpu as pltpu
from jax.experimental.pallas import tpu_sc as plsc

N_BINS = 64
INPUT_DIM = 64
R = 1.2
BOUND = 50.0

NC = 2
NS = 16
NW = NC * NS
CR = 128

_LOG2_POLY = (
    -0.02512326007229515, 0.2700380630042624, -1.2479651030208252,
    3.2494723906595535, -5.301716265875989, 6.089900348117342,
    -3.0346040497219526,
)
_LN2 = float(np.log(2.0))


def _log2_estrin(mant, e_f, scale, bias):
    c6, c5, c4, c3, c2, c1, c0 = [c * scale for c in _LOG2_POLY]
    c0 = c0 + bias * scale
    m2 = mant * mant
    t1 = c1 * mant + c0
    t2 = c3 * mant + c2
    t3 = c5 * mant + c4
    t4 = c6 * m2 + t3
    return (t4 * m2 + t2) * m2 + t1 + e_f * scale


def _static_mesh():
    m = N_BINS / 2
    x1L_raw = BOUND * (R - 1.0) / (np.power(R, m) - 1.0)
    index = np.arange(0, N_BINS + 1, dtype=np.float64).reshape(-1, 1) - m
    xr = np.where(index >= 0, (1.0 - np.power(R, index)) / (1.0 - R),
                  (1.0 - np.power(R, np.abs(index))) / (1.0 - R))
    xr = np.where(index >= 0, x1L_raw * xr, -x1L_raw * xr)
    xr = (xr + BOUND) / 2.0 / BOUND
    mesh = np.concatenate([np.array([[0.0]]), xr[1:-1, :1],
                           np.array([[1.0]])], 0).astype(np.float32)
    elmt = (mesh[1:] - mesh[:-1]).astype(np.float32)
    return mesh, elmt, x1L_raw


_MESH_NP, _ELMT_NP, _X1L_RAW = _static_mesh()
_T_SCALE = np.float32((R - 1.0) / _X1L_RAW)
_INV_L2R = np.float32(1.0 / np.log2(R))

_LUT_NP = ((np.log2(1.0 + (np.arange(1024, dtype=np.float64) + 0.5) / 1024.0)
            - 127.0) / np.log2(R)).astype(np.float32)


def _coeff_tables(p):
    elmt = jnp.asarray(_ELMT_NP)
    p0 = jnp.ones((1, INPUT_DIM), dtype=jnp.float32)
    px = jnp.exp(p) * (elmt[:-1] + elmt[1:]) / 2.0
    px = (1.0 - elmt[0]) / jnp.sum(px, 0, keepdims=True)
    px = px * jnp.exp(p)
    pdf = jnp.concatenate([p0, px, p0], 0)
    cell = (pdf[:-1, :] + pdf[1:, :]) / 2.0 * elmt
    F_ref = jnp.concatenate([jnp.zeros((1, INPUT_DIM), jnp.float32),
                             jnp.cumsum(cell, axis=0)[:N_BINS - 1]], 0)

    v1 = pdf[:N_BINS]
    v2 = pdf[1:N_BINS + 1]
    h = jnp.asarray(_ELMT_NP[:, 0])
    m0x = jnp.asarray(_MESH_NP[:N_BINS, 0] * (2.0 * BOUND) - BOUND)
    c = (v2 - v1) / ((2.0 * 2.0 * BOUND) * h)[:, None]
    b = v1 - (2.0 * m0x)[:, None] * c
    a = ((2.0 * BOUND) * F_ref - BOUND
         + (m0x * m0x)[:, None] * c - m0x[:, None] * v1)
    zrow = jnp.zeros((1, INPUT_DIM), jnp.float32)
    orow = jnp.ones((1, INPUT_DIM), jnp.float32)
    a = jnp.concatenate([zrow, a, zrow], 0)
    b = jnp.concatenate([orow, b, orow], 0)
    c = jnp.concatenate([zrow, c, zrow], 0)
    return a, b, c


def _split_float(a):
    bi = plsc.bitcast(a, jnp.int32)
    e_f = (bi >> 23).astype(jnp.float32)
    mant = plsc.bitcast((bi & 0x007FFFFF) | 0x3F800000, jnp.float32)
    return e_f, mant


def _lnf(a):
    e_f, mant = _split_float(a)
    return _log2_estrin(mant, e_f, _LN2, -127.0)


def _sc_body(batch, x_hbm, a_hbm, b_hbm, c_hbm, l_hbm, y_hbm, ld_hbm,
             xb, yb, ldb, ta, tb, tc, tl,
             sx0, sx1, sy0, sy1, sl0, sl1):
    rows_w = batch // NW
    nch = rows_w // CR
    sx, sy, sl = (sx0, sx1), (sy0, sy1), (sl0, sl1)
    wid = lax.axis_index("s") * NC + lax.axis_index("c")
    pltpu.sync_copy(a_hbm, ta)
    pltpu.sync_copy(b_hbm, tb)
    pltpu.sync_copy(c_hbm, tc)
    pltpu.sync_copy(l_hbm, tl)
    base = wid * rows_w
    lanes = lax.iota(jnp.int32, 16)
    ones = jnp.ones((16,), dtype=jnp.float32)

    def elem(xv, d):
        t = jnp.abs(xv) * _T_SCALE + 1.0
        bi = plsc.bitcast(t, jnp.int32)
        li = (bi >> 13) & 0x3FF
        e_f = (bi >> 23).astype(jnp.float32)
        mf = e_f * _INV_L2R + plsc.load_gather(tl, [li])
        mi = mf.astype(jnp.int32)
        mc = jnp.minimum(jnp.maximum(mi, 0), N_BINS // 2)
        neg = plsc.bitcast(xv, jnp.int32) >> 31
        k = (mc ^ neg) + (N_BINS // 2 + 1)
        tix = k * INPUT_DIM + d
        av = plsc.load_gather(ta, [tix])
        bv = plsc.load_gather(tb, [tix])
        cv = plsc.load_gather(tc, [tix])
        cx = cv * xv
        inner = bv + cx
        y = av + xv * inner
        g = inner + cx
        return y, g

    def x_load(ci, b):
        r0 = pl.multiple_of(base + ci * CR, 8)
        return pltpu.make_async_copy(
            x_hbm.at[pl.ds(r0, CR)], xb.at[pl.ds(b * CR, CR)], sx[b])

    def compute_chunk(ci, b):
        rb = b * CR

        @plsc.parallel_loop(0, CR // 16)
        def rg_body(rg):
            rows = rb + rg * 16 + lanes

            @plsc.parallel_loop(0, INPUT_DIM // 4, unroll=2,
                                carry=(ones, ones, ones, ones))
            def d_body(i, accs):
                a0, a1, a2, a3 = accs
                out = []
                for j, acc in zip(range(4), (a0, a1, a2, a3)):
                    dv = (i * 4 + j + lanes) & (INPUT_DIM - 1)
                    xv = plsc.load_gather(xb, [rows, dv])
                    y, g = elem(xv, dv)
                    plsc.store_scatter(yb, [rows, dv], y)
                    out.append(acc * g)
                return tuple(out)

            a0, a1, a2, a3 = d_body
            ld = _lnf(a0) + _lnf(a1) + _lnf(a2) + _lnf(a3)
            ldb[pl.ds(rb + rg * 16, 16)] = ld

        r0 = pl.multiple_of(base + ci * CR, 8)
        pltpu.make_async_copy(
            yb.at[pl.ds(rb, CR)], y_hbm.at[pl.ds(r0, CR)], sy[b]).start()
        pltpu.make_async_copy(
            ldb.at[pl.ds(rb, CR)], ld_hbm.at[pl.ds(r0, CR)], sl[b]).start()

    x_load(0, 0).start()

    def pair_body(cp, carry):
        for b in range(2):
            ci = cp * 2 + b
            x_load(ci, b).wait()

            @pl.when(ci + 1 < nch)
            def _():
                x_load(ci + 1, 1 - b).start()

            @pl.when(cp > 0)
            def _():
                r2 = pl.multiple_of(base + (ci - 2) * CR, 8)
                pltpu.make_async_copy(
                    yb.at[pl.ds(b * CR, CR)],
                    y_hbm.at[pl.ds(r2, CR)], sy[b]).wait()
                pltpu.make_async_copy(
                    ldb.at[pl.ds(b * CR, CR)],
                    ld_hbm.at[pl.ds(r2, CR)], sl[b]).wait()

            compute_chunk(ci, b)
        return carry

    lax.fori_loop(0, nch // 2, pair_body, 0)
    for b in range(2):
        r2 = pl.multiple_of(base + (nch - 2 + b) * CR, 8)
        pltpu.make_async_copy(
            yb.at[pl.ds(b * CR, CR)], y_hbm.at[pl.ds(r2, CR)], sy[b]).wait()
        pltpu.make_async_copy(
            ldb.at[pl.ds(b * CR, CR)], ld_hbm.at[pl.ds(r2, CR)], sl[b]).wait()


def kernel(x, p):
    batch = x.shape[0]
    a, b, c = _coeff_tables(p)
    mesh = plsc.VectorSubcoreMesh(core_axis_name="c", subcore_axis_name="s")
    run = pl.kernel(
        functools.partial(_sc_body, batch),
        out_type=(jax.ShapeDtypeStruct((batch, INPUT_DIM), jnp.float32),
                  jax.ShapeDtypeStruct((batch,), jnp.float32)),
        mesh=mesh,
        compiler_params=pltpu.CompilerParams(needs_layout_passes=False),
        scratch_types=[
            pltpu.VMEM((2 * CR, INPUT_DIM), jnp.float32),
            pltpu.VMEM((2 * CR, INPUT_DIM), jnp.float32),
            pltpu.VMEM((2 * CR,), jnp.float32),
            pltpu.VMEM(((N_BINS + 2) * INPUT_DIM,), jnp.float32),
            pltpu.VMEM(((N_BINS + 2) * INPUT_DIM,), jnp.float32),
            pltpu.VMEM(((N_BINS + 2) * INPUT_DIM,), jnp.float32),
            pltpu.VMEM((1024,), jnp.float32),
            pltpu.SemaphoreType.DMA,
            pltpu.SemaphoreType.DMA,
            pltpu.SemaphoreType.DMA,
            pltpu.SemaphoreType.DMA,
            pltpu.SemaphoreType.DMA,
            pltpu.SemaphoreType.DMA,
        ],
    )
    y, ld = run(x, a.reshape(-1), b.reshape(-1), c.reshape(-1),
                jnp.asarray(_LUT_NP))
    return (y, ld)
